# baseline (speedup 1.0000x reference)
import functools

import jax
import jax.numpy as jnp
from jax import lax
from jax.experimental import pallas as pl
from jax.experimental.pallas import tpu as pltpu
from jax.experimental.pallas import tpu_sc as plsc

_NC = 2
_NS = 16
_NW = _NC * _NS
_LANES = 16


def _table_body(emb_ref, wt_ref, b_ref, *out_refs):
    t = (
        jnp.dot(emb_ref[...], wt_ref[...], preferred_element_type=jnp.float32)
        + b_ref[...]
    )
    q = t.shape[1] // len(out_refs)
    for i, o in enumerate(out_refs):
        o[...] = t[:, i * q:(i + 1) * q]


def _make_logits_table(emb, Wt, b2):
    V = emb.shape[0]
    Q = Wt.shape[1] // 4
    return pl.pallas_call(
        _table_body,
        out_shape=[jax.ShapeDtypeStruct((V, Q), jnp.float32)] * 4,
    )(emb, Wt, b2)


def _make_gather(L, V, Vp, B):
    Q = Vp // 4
    NBLK_B = B // 128
    NU = L * NBLK_B
    KMAX = (NU + _NW - 1) // _NW
    ROWS = [128, 128, 128, 128, 128, 128, 128, V - 7 * 128]

    mesh = plsc.VectorSubcoreMesh(
        core_axis_name="c", subcore_axis_name="s",
        num_cores=_NC, num_subcores=_NS,
    )

    @functools.partial(
        pl.kernel,
        out_type=jax.ShapeDtypeStruct((L, V, B), jnp.float32),
        mesh=mesh,
        scratch_types=[
            pltpu.VMEM((2, 128), jnp.int32),
            pltpu.VMEM((2, 128, Q), jnp.float32),
            pltpu.VMEM((2, 128, 128), jnp.float32),
            pltpu.SemaphoreType.DMA,
            pltpu.SemaphoreType.DMA,
            pltpu.SemaphoreType.DMA,
            pltpu.SemaphoreType.DMA,
        ],
        compiler_params=pltpu.CompilerParams(
            use_tc_tiling_on_sc=True, needs_layout_passes=False
        ),
    )
    def gather(
        t0, t1, t2, t3, xt_hbm, out_hbm, idx_v, G, S, gsem, isem, o0, o1
    ):
        wid = lax.axis_index("s") * _NC + lax.axis_index("c")
        osem = (o0, o1)
        tabs = (t0, t1, t2, t3)
        iota = lax.iota(jnp.int32, 16)
        skews = [(iota + d) & 15 for d in range(16)]

        def wait_out(s, rows):
            pltpu.make_async_copy(
                S.at[s, pl.ds(0, rows)],
                out_hbm.at[0, pl.ds(0, rows), pl.ds(0, 128)],
                osem[s],
            ).wait()

        def wait_gather(q):
            pltpu.make_async_copy(
                tabs[q].at[pl.ds(0, 128)], G.at[q & 1], gsem
            ).wait()

        def unit_body(k, carry):
            u = wid + _NW * k
            kp = k & 1

            @pl.when(u < NU)
            def _():
                l = u // NBLK_B
                blk = u % NBLK_B
                b0 = pl.multiple_of(blk * 128, 128)
                un = u + _NW
                for q in range(4):
                    g = q & 1
                    wait_gather(q)
                    if q < 3:
                        pltpu.async_copy(
                            tabs[q + 1].at[idx_v.at[kp]],
                            G.at[(q + 1) & 1],
                            gsem,
                        )
                    if q == 1:
                        @pl.when(un < NU)
                        def _():
                            ln = un // NBLK_B
                            bn = un % NBLK_B
                            pltpu.async_copy(
                                xt_hbm.at[ln, bn], idx_v.at[1 - kp], isem
                            )
                    if q == 3:
                        @pl.when(un < NU)
                        def _():
                            pltpu.make_async_copy(
                                xt_hbm.at[0, 0], idx_v.at[1 - kp], isem
                            ).wait()
                            pltpu.async_copy(
                                tabs[0].at[idx_v.at[1 - kp]], G.at[0], gsem
                            )
                    for vbi in range(2):
                        sblk = 2 * q + vbi
                        rows = ROWS[sblk]
                        s = sblk & 1
                        prev_rows = ROWS[(sblk - 2) % 8]
                        if sblk >= 2:
                            wait_out(s, prev_rows)
                        else:
                            @pl.when(k > 0)
                            def _():
                                wait_out(s, prev_rows)

                        @plsc.parallel_loop(0, 64, 1, unroll=4)
                        def sb_loop(sb, g=g, vbi=vbi, s=s):
                            cs = sb // 8
                            bs = sb % 8
                            rws = iota + bs * 16
                            for d in range(16):
                                colsG = skews[d] + (vbi * 128 + cs * 16)
                                colsS = skews[d] + cs * 16
                                val = plsc.load_gather(
                                    G.at[g], [rws, colsG]
                                )
                                plsc.store_scatter(
                                    S.at[s], [colsS, rws], val
                                )
                        pltpu.async_copy(
                            S.at[s, pl.ds(0, rows)],
                            out_hbm.at[
                                l,
                                pl.ds(sblk * 128, rows),
                                pl.ds(b0, 128),
                            ],
                            osem[s],
                        )
            return carry

        pltpu.sync_copy(xt_hbm.at[wid // NBLK_B, wid % NBLK_B], idx_v.at[0])
        pltpu.async_copy(tabs[0].at[idx_v.at[0]], G.at[0], gsem)
        lax.fori_loop(0, KMAX, unit_body, 0)
        wait_out(0, ROWS[6])
        wait_out(1, ROWS[7])

    return gather


def kernel(x, emb, W, b):
    V, D = emb.shape
    B, L = x.shape

    Vp = (V + 255) // 256 * 256
    Wt = jnp.pad(W.T, ((0, 0), (0, Vp - V)))
    b2 = jnp.pad(b, (0, Vp - V)).reshape(1, Vp)
    t0, t1, t2, t3 = _make_logits_table(emb, Wt, b2)

    xt = x.T.reshape(L, B // 128, 128).astype(jnp.int32)
    p = _make_gather(L, V, Vp, B)(t0, t1, t2, t3, xt)
    return jnp.transpose(p, (2, 0, 1))


# speedup vs baseline: 1.1728x; 1.1728x over previous
import functools

import jax
import jax.numpy as jnp
from jax import lax
from jax.experimental import pallas as pl
from jax.experimental.pallas import tpu as pltpu
from jax.experimental.pallas import tpu_sc as plsc

_NC = 2
_NS = 16
_NW = _NC * _NS
_LANES = 16


def _table_body(emb_ref, wt_ref, b_ref, *out_refs):
    t = (
        jnp.dot(emb_ref[...], wt_ref[...], preferred_element_type=jnp.float32)
        + b_ref[...]
    )
    V = t.shape[1]
    q = out_refs[0].shape[1]
    for i, o in enumerate(out_refs):
        lo = i * q
        if lo + q <= V:
            o[...] = t[:, lo:lo + q]
        else:
            o[...] = jnp.pad(t[:, lo:V], ((0, 0), (0, lo + q - V)))


def _make_logits_table(emb, Wt, b2, Vp):
    V = emb.shape[0]
    Q = Vp // 4
    return pl.pallas_call(
        _table_body,
        out_shape=[jax.ShapeDtypeStruct((V, Q), jnp.float32)] * 4,
    )(emb, Wt, b2)


def _make_gather(L, V, Vp, B):
    Q = Vp // 4
    NBLK_B = B // 128
    NU = L * NBLK_B
    KMAX = (NU + _NW - 1) // _NW
    ROWS = [128, 128, 128, 128, 128, 128, 128, V - 7 * 128]

    mesh = plsc.VectorSubcoreMesh(
        core_axis_name="c", subcore_axis_name="s",
        num_cores=_NC, num_subcores=_NS,
    )

    @functools.partial(
        pl.kernel,
        out_type=jax.ShapeDtypeStruct((L, V, B), jnp.float32),
        mesh=mesh,
        scratch_types=[
            pltpu.VMEM((2, 128), jnp.int32),
            pltpu.VMEM((2, 128, Q), jnp.float32),
            pltpu.VMEM((2, 128, 128), jnp.float32),
            pltpu.SemaphoreType.DMA,
            pltpu.SemaphoreType.DMA,
            pltpu.SemaphoreType.DMA,
            pltpu.SemaphoreType.DMA,
        ],
        compiler_params=pltpu.CompilerParams(
            use_tc_tiling_on_sc=True, needs_layout_passes=False
        ),
    )
    def gather(
        t0, t1, t2, t3, xt_hbm, out_hbm, idx_v, G, S,
        gsem, isem, o0, o1
    ):
        wid = lax.axis_index("s") * _NC + lax.axis_index("c")
        osem = (o0, o1)
        tabs = (t0, t1, t2, t3)
        shs = tabs
        iota = lax.iota(jnp.int32, 16)
        skews = [(iota + d) & 15 for d in range(16)]

        def wait_out(s, rows):
            pltpu.make_async_copy(
                S.at[s, pl.ds(0, rows)],
                out_hbm.at[0, pl.ds(0, rows), pl.ds(0, 128)],
                osem[s],
            ).wait()

        def wait_gather(q):
            pltpu.make_async_copy(
                tabs[q].at[pl.ds(0, 128)], G.at[q & 1], gsem
            ).wait()

        def unit_body(k, carry):
            u = wid + _NW * k
            kp = k & 1

            @pl.when(u < NU)
            def _():
                l = u // NBLK_B
                blk = u % NBLK_B
                b0 = pl.multiple_of(blk * 128, 128)
                un = u + _NW
                for q in range(4):
                    g = q & 1
                    wait_gather(q)
                    if q < 3:
                        pltpu.async_copy(
                            shs[q + 1].at[idx_v.at[kp]],
                            G.at[(q + 1) & 1],
                            gsem,
                        )
                    if q == 1:
                        @pl.when(un < NU)
                        def _():
                            ln = un // NBLK_B
                            bn = un % NBLK_B
                            pltpu.async_copy(
                                xt_hbm.at[ln, bn], idx_v.at[1 - kp], isem
                            )
                    if q == 3:
                        @pl.when(un < NU)
                        def _():
                            pltpu.make_async_copy(
                                xt_hbm.at[0, 0], idx_v.at[1 - kp], isem
                            ).wait()
                            pltpu.async_copy(
                                shs[0].at[idx_v.at[1 - kp]], G.at[0], gsem
                            )
                    for vbi in range(2):
                        sblk = 2 * q + vbi
                        rows = ROWS[sblk]
                        s = sblk & 1
                        prev_rows = ROWS[(sblk - 2) % 8]
                        if sblk >= 2:
                            wait_out(s, prev_rows)
                        else:
                            @pl.when(k > 0)
                            def _():
                                wait_out(s, prev_rows)

                        @plsc.parallel_loop(0, 64, 1, unroll=2)
                        def sb_loop(sb, g=g, vbi=vbi, s=s):
                            cs = sb // 8
                            bs = sb % 8
                            rws = iota + bs * 16
                            for d in range(16):
                                colsG = skews[d] + (vbi * 128 + cs * 16)
                                colsS = skews[d] + cs * 16
                                val = plsc.load_gather(
                                    G.at[g], [rws, colsG]
                                )
                                plsc.store_scatter(
                                    S.at[s], [colsS, rws], val
                                )
                        pltpu.async_copy(
                            S.at[s, pl.ds(0, rows)],
                            out_hbm.at[
                                l,
                                pl.ds(sblk * 128, rows),
                                pl.ds(b0, 128),
                            ],
                            osem[s],
                        )
            return carry

        pltpu.sync_copy(xt_hbm.at[wid // NBLK_B, wid % NBLK_B], idx_v.at[0])
        pltpu.async_copy(shs[0].at[idx_v.at[0]], G.at[0], gsem)
        lax.fori_loop(0, KMAX, unit_body, 0)
        wait_out(0, ROWS[6])
        wait_out(1, ROWS[7])

    return gather


def kernel(x, emb, W, b):
    V, D = emb.shape
    B, L = x.shape

    Vp = (V + 255) // 256 * 256
    t0, t1, t2, t3 = _make_logits_table(emb, W.T, b.reshape(1, V), Vp)

    xt = x.T.reshape(L, B // 128, 128).astype(jnp.int32)
    p = _make_gather(L, V, Vp, B)(t0, t1, t2, t3, xt)
    return jnp.transpose(p, (2, 0, 1))
